# baseline (speedup 1.0000x reference)
import jax
import jax.numpy as jnp
from jax import lax
from jax.experimental import pallas as pl
from jax.experimental.pallas import tpu as pltpu
from jax.experimental.pallas import tpu_sc as plsc

N = 10000
NP = 10240
E = 320000
D_IN = 128
D_HID = 128
D_OUT = 64

NC, NS = 2, 16
NW = NC * NS
CHUNK = 80
EPW = E // NW
CPW = EPW // CHUNK
EPT = E // NS
CPT = EPT // CHUNK
RPT = NP // NS
CHUNKD = 128
CPTD = EPT // CHUNKD
TAILD2 = EPT % CHUNKD
GRPD2 = CPTD // 4
NBUF = 4
GRP = CPW // NBUF
TAIL = CPW % NBUF
GRPD = CPT // NBUF
TAILD = CPT % NBUF

_mesh = plsc.VectorSubcoreMesh(
    core_axis_name="c", subcore_axis_name="s", num_cores=NC, num_subcores=NS)


def _deg_body(edges_hbm, ones_hbm, zeros_hbm, out_hbm,
              acc, idx_v, tidx_v, ones_v, semi0, semi1, sems):
    c = lax.axis_index("c")
    s = lax.axis_index("s")
    pltpu.sync_copy(zeros_hbm.at[pl.ds(s * RPT, RPT)],
                    acc.at[pl.ds(s * RPT, RPT)])
    pltpu.sync_copy(ones_hbm, ones_v)
    plsc.subcore_barrier()

    base = c * E + s * EPT
    semi = (semi0, semi1)

    def fire_idx(g, p):
        for b in range(NBUF):
            raw = base + (g * NBUF + b) * CHUNKD
            st = pl.multiple_of(jnp.minimum(raw, 2 * E - CHUNKD), 8)
            pltpu.async_copy(edges_hbm.at[pl.ds(st, CHUNKD)],
                             idx_v.at[p, b], semi[p])

    def drain_scatters(p):
        for b in range(NBUF):
            pltpu.make_async_copy(ones_v, acc.at[idx_v.at[p, b]],
                                  sems).wait()

    def work(g, p, next_g, drain_prev):
        for b in range(NBUF):
            pltpu.make_async_copy(edges_hbm.at[pl.ds(0, CHUNKD)],
                                  idx_v.at[p, b], semi[p]).wait()
        if drain_prev:
            drain_scatters(p)
        if next_g is not None:
            fire_idx(next_g, 1 - p)
        for b in range(NBUF):
            pltpu.async_copy(ones_v, acc.at[idx_v.at[p, b]], sems,
                             add=True)

    fire_idx(0, 0)
    work(0, 0, 1, False)

    def pair(m, carry):
        work(2 * m + 1, 1, 2 * m + 2, True)
        work(2 * m + 2, 0, 2 * m + 3, True)
        return carry

    lax.fori_loop(0, (GRPD2 - 1) // 2 - 1, pair, 0)
    work(GRPD2 - 2, 1, GRPD2 - 1, True)
    work(GRPD2 - 1, 0, None, True)
    drain_scatters(0)
    st = pl.multiple_of(base + CPTD * CHUNKD, 8)
    pltpu.sync_copy(edges_hbm.at[pl.ds(st, TAILD2)], tidx_v)
    pltpu.sync_copy(ones_v.at[pl.ds(0, TAILD2)], acc.at[tidx_v], add=True)

    plsc.subcore_barrier()
    pltpu.sync_copy(acc.at[pl.ds(s * RPT, RPT)],
                    out_hbm.at[c, pl.ds(s * RPT, RPT)])


_deg_call = pl.kernel(
    _deg_body,
    out_type=jax.ShapeDtypeStruct((NC, NP, 128), jnp.float32),
    mesh=_mesh,
    scratch_types=[
        pltpu.VMEM_SHARED((NP, 128), jnp.float32),
        pltpu.VMEM((2, NBUF, CHUNKD), jnp.int32),
        pltpu.VMEM((TAILD2,), jnp.int32),
        pltpu.VMEM((CHUNKD, 128), jnp.float32),
        pltpu.SemaphoreType.DMA,
        pltpu.SemaphoreType.DMA,
        pltpu.SemaphoreType.DMA,
    ],
)


def _scatter_body(pre_hbm, src_hbm, dst_hbm, zeros_hbm, out_hbm,
                  acc, idx_s, idx_d, rows,
                  semi0, semi1, semg0, semg1, semg2, semg3, sems):
    c = lax.axis_index("c")
    s = lax.axis_index("s")
    w = s * NC + c
    pltpu.sync_copy(zeros_hbm.at[pl.ds(s * RPT, RPT)],
                    acc.at[pl.ds(s * RPT, RPT)])
    plsc.subcore_barrier()

    base = w * EPW
    semi = (semi0, semi1)
    semg = (semg0, semg1, semg2, semg3)

    def fire_idx(g, p):
        for b in range(NBUF):
            raw = base + (g * NBUF + b) * CHUNK
            st = pl.multiple_of(jnp.minimum(raw, E - CHUNK), 8)
            pltpu.async_copy(src_hbm.at[pl.ds(st, CHUNK)],
                             idx_s.at[p, b], semi[p])
            pltpu.async_copy(dst_hbm.at[pl.ds(st, CHUNK)],
                             idx_d.at[p, b], semi[p])

    def drain_scatters(p):
        for b in range(NBUF):
            pltpu.make_async_copy(rows.at[b], acc.at[idx_d.at[p, b]],
                                  sems).wait()

    def work(g, p, next_g, drain_prev):
        for b in range(NBUF):
            pltpu.make_async_copy(src_hbm.at[pl.ds(0, CHUNK)],
                                  idx_s.at[p, b], semi[p]).wait()
            pltpu.make_async_copy(dst_hbm.at[pl.ds(0, CHUNK)],
                                  idx_d.at[p, b], semi[p]).wait()
        if drain_prev:
            drain_scatters(p)
        if next_g is not None:
            fire_idx(next_g, 1 - p)
        dg = [pltpu.async_copy(pre_hbm.at[idx_s.at[p, b]], rows.at[b],
                               semg[b])
              for b in range(NBUF)]
        for b in range(NBUF):
            dg[b].wait()
            pltpu.async_copy(rows.at[b], acc.at[idx_d.at[p, b]],
                             sems, add=True)

    fire_idx(0, 0)
    work(0, 0, 1, False)

    def pair(m, carry):
        work(2 * m + 1, 1, 2 * m + 2, True)
        work(2 * m + 2, 0, 2 * m + 3, True)
        return carry

    lax.fori_loop(0, GRP // 2 - 1, pair, 0)
    work(GRP - 2, 1, GRP - 1, True)
    work(GRP - 1, 0, None, True)
    drain_scatters(0)
    for t in range(TAIL):
        st = pl.multiple_of(base + (GRP * NBUF + t) * CHUNK, 8)
        pltpu.sync_copy(src_hbm.at[pl.ds(st, CHUNK)], idx_s.at[0, 0])
        pltpu.sync_copy(dst_hbm.at[pl.ds(st, CHUNK)], idx_d.at[0, 0])
        pltpu.async_copy(pre_hbm.at[idx_s.at[0, 0]], rows.at[0],
                         semg[0]).wait()
        pltpu.sync_copy(rows.at[0], acc.at[idx_d.at[0, 0]], add=True)

    plsc.subcore_barrier()
    pltpu.sync_copy(acc.at[pl.ds(s * RPT, RPT)],
                    out_hbm.at[c, pl.ds(s * RPT, RPT)])


_scatter128 = pl.kernel(
    _scatter_body,
    out_type=jax.ShapeDtypeStruct((NC, NP, 128), jnp.float32),
    mesh=_mesh,
    scratch_types=[
        pltpu.VMEM_SHARED((NP, 128), jnp.float32),
        pltpu.VMEM((2, NBUF, CHUNK), jnp.int32),
        pltpu.VMEM((2, NBUF, CHUNK), jnp.int32),
        pltpu.VMEM((NBUF, CHUNK, 128), jnp.float32),
        pltpu.SemaphoreType.DMA,
        pltpu.SemaphoreType.DMA,
        pltpu.SemaphoreType.DMA,
        pltpu.SemaphoreType.DMA,
        pltpu.SemaphoreType.DMA,
        pltpu.SemaphoreType.DMA,
        pltpu.SemaphoreType.DMA,
    ],
)


_BM = 1000


def _rs(deg_ref):
    return lax.rsqrt(jnp.maximum(deg_ref[0, :, 0:1], 1.0))


def _deg_spec(k):
    return pl.BlockSpec((1, _BM, 128), lambda i: (k, i, 0))


def _mm_scale_body(x_ref, w_ref, dego_ref, o_ref):
    o_ref[...] = jnp.dot(x_ref[...], w_ref[...],
                         preferred_element_type=jnp.float32) * _rs(dego_ref)


def _mm_scale(x, w, degs):
    din, dout = w.shape
    return pl.pallas_call(
        _mm_scale_body,
        grid=(N // _BM,),
        in_specs=[
            pl.BlockSpec((_BM, din), lambda i: (i, 0)),
            pl.BlockSpec((din, dout), lambda i: (0, 0)),
            _deg_spec(0),
        ],
        out_specs=pl.BlockSpec((_BM, dout), lambda i: (i, 0)),
        out_shape=jax.ShapeDtypeStruct((N, dout), jnp.float32),
    )(x, w, degs)


def _combine_mm_body(p_ref, degi_ref, b_ref, w_ref, dego_ref, o_ref):
    h = (p_ref[0] + p_ref[1]) * _rs(degi_ref) + b_ref[...]
    h = jnp.maximum(h, 0.0)
    o_ref[...] = jnp.dot(h, w_ref[...],
                         preferred_element_type=jnp.float32) * _rs(dego_ref)


def _combine_mm(parts, degs, b, w):
    din, dout = w.shape
    return pl.pallas_call(
        _combine_mm_body,
        grid=(N // _BM,),
        in_specs=[
            pl.BlockSpec((NC, _BM, din), lambda i: (0, i, 0)),
            _deg_spec(1),
            pl.BlockSpec((1, din), lambda i: (0, 0)),
            pl.BlockSpec((din, dout), lambda i: (0, 0)),
            _deg_spec(0),
        ],
        out_specs=pl.BlockSpec((_BM, dout), lambda i: (i, 0)),
        out_shape=jax.ShapeDtypeStruct((NP, dout), jnp.float32),
    )(parts, degs, b, w, degs)


def _final_body(p_ref, degi_ref, b_ref, o_ref):
    v = (p_ref[0] + p_ref[1])[:, :D_OUT]
    o_ref[...] = v * _rs(degi_ref) + b_ref[...]


def _final(parts, degs, b):
    return pl.pallas_call(
        _final_body,
        grid=(N // _BM,),
        in_specs=[
            pl.BlockSpec((NC, _BM, D_HID), lambda i: (0, i, 0)),
            _deg_spec(1),
            pl.BlockSpec((1, D_OUT), lambda i: (0, 0)),
        ],
        out_specs=pl.BlockSpec((_BM, D_OUT), lambda i: (i, 0)),
        out_shape=jax.ShapeDtypeStruct((N, D_OUT), jnp.float32),
    )(parts, degs, b)


@jax.jit
def kernel(x, edge_index, W1, b1, W2, b2):
    src = edge_index[0]
    dst = edge_index[1]
    edges_flat = edge_index.reshape(-1)

    ones128 = jnp.ones((CHUNKD, 128), jnp.float32)
    zeros128 = jnp.zeros((NP, 128), jnp.float32)
    degs = _deg_call(edges_flat, ones128, zeros128)

    pre1 = _mm_scale(x, W1, degs)
    part1 = _scatter128(pre1, src, dst, zeros128)
    W2p = jnp.pad(W2, ((0, 0), (0, D_HID - D_OUT)))
    pre2 = _combine_mm(part1, degs, b1.reshape(1, D_HID), W2p)
    part2 = _scatter128(pre2, src, dst, zeros128)
    return _final(part2, degs, b2.reshape(1, D_OUT))


# speedup vs baseline: 1.1094x; 1.1094x over previous
import jax
import jax.numpy as jnp
from jax import lax
from jax.experimental import pallas as pl
from jax.experimental.pallas import tpu as pltpu
from jax.experimental.pallas import tpu_sc as plsc

N = 10000
NP = 10240
E = 320000
D_IN = 128
D_HID = 128
D_OUT = 64

NC, NS = 2, 16
NW = NC * NS
CHUNK = 80
EPW = E // NW
CPW = EPW // CHUNK
EPT = E // NS
CPT = EPT // CHUNK
RPT = NP // NS
CHUNKD = 128
CPTD = EPT // CHUNKD
TAILD2 = EPT % CHUNKD
GRPD2 = CPTD // 4
NBUF = 4
GRP = CPW // NBUF
TAIL = CPW % NBUF
GRPD = CPT // NBUF
TAILD = CPT % NBUF

_mesh = plsc.VectorSubcoreMesh(
    core_axis_name="c", subcore_axis_name="s", num_cores=NC, num_subcores=NS)


def _deg_body(edges_hbm, ones_hbm, zeros_hbm, out_hbm,
              acc, idx_v, tidx_v, ones_v, semi0, semi1, sems):
    c = lax.axis_index("c")
    s = lax.axis_index("s")
    pltpu.sync_copy(zeros_hbm.at[pl.ds(s * RPT, RPT)],
                    acc.at[pl.ds(s * RPT, RPT)])
    pltpu.sync_copy(ones_hbm, ones_v)
    plsc.subcore_barrier()

    base = c * E + s * EPT
    semi = (semi0, semi1)

    def fire_idx(g, p):
        for b in range(NBUF):
            raw = base + (g * NBUF + b) * CHUNKD
            st = pl.multiple_of(jnp.minimum(raw, 2 * E - CHUNKD), 8)
            pltpu.async_copy(edges_hbm.at[pl.ds(st, CHUNKD)],
                             idx_v.at[p, b], semi[p])

    def drain_scatters(p):
        for b in range(NBUF):
            pltpu.make_async_copy(ones_v, acc.at[idx_v.at[p, b]],
                                  sems).wait()

    def work(g, p, next_g, drain_prev):
        for b in range(NBUF):
            pltpu.make_async_copy(edges_hbm.at[pl.ds(0, CHUNKD)],
                                  idx_v.at[p, b], semi[p]).wait()
        if drain_prev:
            drain_scatters(p)
        if next_g is not None:
            fire_idx(next_g, 1 - p)
        for b in range(NBUF):
            pltpu.async_copy(ones_v, acc.at[idx_v.at[p, b]], sems,
                             add=True)

    fire_idx(0, 0)
    work(0, 0, 1, False)

    def pair(m, carry):
        work(2 * m + 1, 1, 2 * m + 2, True)
        work(2 * m + 2, 0, 2 * m + 3, True)
        return carry

    lax.fori_loop(0, (GRPD2 - 1) // 2 - 1, pair, 0)
    work(GRPD2 - 2, 1, GRPD2 - 1, True)
    work(GRPD2 - 1, 0, None, True)
    drain_scatters(0)
    st = pl.multiple_of(base + CPTD * CHUNKD, 8)
    pltpu.sync_copy(edges_hbm.at[pl.ds(st, TAILD2)], tidx_v)
    pltpu.sync_copy(ones_v.at[pl.ds(0, TAILD2)], acc.at[tidx_v], add=True)

    plsc.subcore_barrier()
    pltpu.sync_copy(acc.at[pl.ds(s * RPT, RPT)],
                    out_hbm.at[c, pl.ds(s * RPT, RPT)])


_deg_call = pl.kernel(
    _deg_body,
    out_type=jax.ShapeDtypeStruct((NC, NP, 128), jnp.float32),
    mesh=_mesh,
    scratch_types=[
        pltpu.VMEM_SHARED((NP, 128), jnp.float32),
        pltpu.VMEM((2, NBUF, CHUNKD), jnp.int32),
        pltpu.VMEM((TAILD2,), jnp.int32),
        pltpu.VMEM((CHUNKD, 128), jnp.float32),
        pltpu.SemaphoreType.DMA,
        pltpu.SemaphoreType.DMA,
        pltpu.SemaphoreType.DMA,
    ],
)


def _scatter_body(pre_hbm, src_hbm, dst_hbm, zeros_hbm, out_hbm,
                  acc, idx_s, idx_d, rows,
                  semi0, semi1, semg0, semg1, semg2, semg3, sems):
    c = lax.axis_index("c")
    s = lax.axis_index("s")
    w = s * NC + c
    pltpu.sync_copy(zeros_hbm.at[pl.ds(s * RPT, RPT)],
                    acc.at[pl.ds(s * RPT, RPT)])
    plsc.subcore_barrier()

    base = w * EPW
    semi = (semi0, semi1)
    semg = (semg0, semg1, semg2, semg3)

    def fire_idx(g, p):
        for b in range(NBUF):
            raw = base + (g * NBUF + b) * CHUNK
            st = pl.multiple_of(jnp.minimum(raw, E - CHUNK), 8)
            pltpu.async_copy(src_hbm.at[pl.ds(st, CHUNK)],
                             idx_s.at[p, b], semi[p])
            pltpu.async_copy(dst_hbm.at[pl.ds(st, CHUNK)],
                             idx_d.at[p, b], semi[p])

    def drain_scatters(p):
        for b in range(NBUF):
            pltpu.make_async_copy(rows.at[b], acc.at[idx_d.at[p, b]],
                                  sems).wait()

    def work(g, p, next_g, drain_prev):
        for b in range(NBUF):
            pltpu.make_async_copy(src_hbm.at[pl.ds(0, CHUNK)],
                                  idx_s.at[p, b], semi[p]).wait()
            pltpu.make_async_copy(dst_hbm.at[pl.ds(0, CHUNK)],
                                  idx_d.at[p, b], semi[p]).wait()
        if drain_prev:
            drain_scatters(p)
        if next_g is not None:
            fire_idx(next_g, 1 - p)
        dg = [pltpu.async_copy(pre_hbm.at[idx_s.at[p, b]], rows.at[b],
                               semg[b])
              for b in range(NBUF)]
        for b in range(NBUF):
            dg[b].wait()
            pltpu.async_copy(rows.at[b], acc.at[idx_d.at[p, b]],
                             sems, add=True)

    fire_idx(0, 0)
    work(0, 0, 1, False)

    def pair(m, carry):
        work(2 * m + 1, 1, 2 * m + 2, True)
        work(2 * m + 2, 0, 2 * m + 3, True)
        return carry

    lax.fori_loop(0, GRP // 2 - 1, pair, 0)
    work(GRP - 2, 1, GRP - 1, True)
    work(GRP - 1, 0, None, True)
    drain_scatters(0)
    for t in range(TAIL):
        st = pl.multiple_of(base + (GRP * NBUF + t) * CHUNK, 8)
        pltpu.sync_copy(src_hbm.at[pl.ds(st, CHUNK)], idx_s.at[0, 0])
        pltpu.sync_copy(dst_hbm.at[pl.ds(st, CHUNK)], idx_d.at[0, 0])
        pltpu.async_copy(pre_hbm.at[idx_s.at[0, 0]], rows.at[0],
                         semg[0]).wait()
        pltpu.sync_copy(rows.at[0], acc.at[idx_d.at[0, 0]], add=True)

    plsc.subcore_barrier()
    pltpu.sync_copy(acc.at[pl.ds(s * RPT, RPT)],
                    out_hbm.at[c, pl.ds(s * RPT, RPT)])


_scatter128 = pl.kernel(
    _scatter_body,
    out_type=jax.ShapeDtypeStruct((NC, NP, 128), jnp.float32),
    mesh=_mesh,
    scratch_types=[
        pltpu.VMEM_SHARED((NP, 128), jnp.float32),
        pltpu.VMEM((2, NBUF, CHUNK), jnp.int32),
        pltpu.VMEM((2, NBUF, CHUNK), jnp.int32),
        pltpu.VMEM((NBUF, CHUNK, 128), jnp.float32),
        pltpu.SemaphoreType.DMA,
        pltpu.SemaphoreType.DMA,
        pltpu.SemaphoreType.DMA,
        pltpu.SemaphoreType.DMA,
        pltpu.SemaphoreType.DMA,
        pltpu.SemaphoreType.DMA,
        pltpu.SemaphoreType.DMA,
    ],
)


_scatter64 = pl.kernel(
    _scatter_body,
    out_type=jax.ShapeDtypeStruct((NC, NP, D_OUT), jnp.float32),
    mesh=_mesh,
    compiler_params=pltpu.CompilerParams(use_tc_tiling_on_sc=False),
    scratch_types=[
        pltpu.VMEM_SHARED((NP, D_OUT), jnp.float32),
        pltpu.VMEM((2, NBUF, CHUNK), jnp.int32),
        pltpu.VMEM((2, NBUF, CHUNK), jnp.int32),
        pltpu.VMEM((NBUF, CHUNK, D_OUT), jnp.float32),
        pltpu.SemaphoreType.DMA,
        pltpu.SemaphoreType.DMA,
        pltpu.SemaphoreType.DMA,
        pltpu.SemaphoreType.DMA,
        pltpu.SemaphoreType.DMA,
        pltpu.SemaphoreType.DMA,
        pltpu.SemaphoreType.DMA,
    ],
)


_BM = 1000


def _rs(deg_ref):
    return lax.rsqrt(jnp.maximum(deg_ref[0, :, 0:1], 1.0))


def _deg_spec(k):
    return pl.BlockSpec((1, _BM, 128), lambda i: (k, i, 0))


def _mm_scale_body(x_ref, w_ref, dego_ref, o_ref):
    o_ref[...] = jnp.dot(x_ref[...], w_ref[...],
                         preferred_element_type=jnp.float32) * _rs(dego_ref)


def _mm_scale(x, w, degs):
    din, dout = w.shape
    return pl.pallas_call(
        _mm_scale_body,
        grid=(N // _BM,),
        in_specs=[
            pl.BlockSpec((_BM, din), lambda i: (i, 0)),
            pl.BlockSpec((din, dout), lambda i: (0, 0)),
            _deg_spec(0),
        ],
        out_specs=pl.BlockSpec((_BM, dout), lambda i: (i, 0)),
        out_shape=jax.ShapeDtypeStruct((N, dout), jnp.float32),
    )(x, w, degs)


def _combine_mm_body(p_ref, degi_ref, b_ref, w_ref, dego_ref, o_ref):
    h = (p_ref[0] + p_ref[1]) * _rs(degi_ref) + b_ref[...]
    h = jnp.maximum(h, 0.0)
    o_ref[...] = jnp.dot(h, w_ref[...],
                         preferred_element_type=jnp.float32) * _rs(dego_ref)


def _combine_mm(parts, degs, b, w):
    din, dout = w.shape
    return pl.pallas_call(
        _combine_mm_body,
        grid=(N // _BM,),
        in_specs=[
            pl.BlockSpec((NC, _BM, din), lambda i: (0, i, 0)),
            _deg_spec(1),
            pl.BlockSpec((1, din), lambda i: (0, 0)),
            pl.BlockSpec((din, dout), lambda i: (0, 0)),
            _deg_spec(0),
        ],
        out_specs=pl.BlockSpec((_BM, dout), lambda i: (i, 0)),
        out_shape=jax.ShapeDtypeStruct((NP, dout), jnp.float32),
    )(parts, degs, b, w, degs)


def _final_body(p_ref, degi_ref, b_ref, o_ref):
    v = p_ref[0] + p_ref[1]
    o_ref[...] = v * _rs(degi_ref) + b_ref[...]


def _final(parts, degs, b):
    return pl.pallas_call(
        _final_body,
        grid=(N // _BM,),
        in_specs=[
            pl.BlockSpec((NC, _BM, D_OUT), lambda i: (0, i, 0)),
            _deg_spec(1),
            pl.BlockSpec((1, D_OUT), lambda i: (0, 0)),
        ],
        out_specs=pl.BlockSpec((_BM, D_OUT), lambda i: (i, 0)),
        out_shape=jax.ShapeDtypeStruct((N, D_OUT), jnp.float32),
    )(parts, degs, b)


@jax.jit
def kernel(x, edge_index, W1, b1, W2, b2):
    src = edge_index[0]
    dst = edge_index[1]
    edges_flat = edge_index.reshape(-1)

    ones128 = jnp.ones((CHUNKD, 128), jnp.float32)
    zeros128 = jnp.zeros((NP, 128), jnp.float32)
    degs = _deg_call(edges_flat, ones128, zeros128)

    pre1 = _mm_scale(x, W1, degs)
    part1 = _scatter128(pre1, src, dst, zeros128)
    pre2 = _combine_mm(part1, degs, b1.reshape(1, D_HID), W2)
    zeros64 = jnp.zeros((NP, D_OUT), jnp.float32)
    part2 = _scatter64(pre2, src, dst, zeros64)
    return _final(part2, degs, b2.reshape(1, D_OUT))


# speedup vs baseline: 1.3950x; 1.2574x over previous
import jax
import jax.numpy as jnp
from jax import lax
from jax.experimental import pallas as pl
from jax.experimental.pallas import tpu as pltpu
from jax.experimental.pallas import tpu_sc as plsc

N = 10000
NP = 10240
E = 320000
D_IN = 128
D_HID = 128
D_OUT = 64

NC, NS = 2, 16
NW = NC * NS
CHUNK = 80
EPW = E // NW
CPW = EPW // CHUNK
EPT = E // NS
CPT = EPT // CHUNK
RPT = NP // NS
CHUNKD = 128
CPTD = EPT // CHUNKD
TAILD2 = EPT % CHUNKD
GRPD2 = CPTD // 4
NBUF = 4
GRP = CPW // NBUF
TAIL = CPW % NBUF
GRPD = CPT // NBUF
TAILD = CPT % NBUF

_mesh = plsc.VectorSubcoreMesh(
    core_axis_name="c", subcore_axis_name="s", num_cores=NC, num_subcores=NS)


def _deg_body(edges_hbm, ones_hbm, zeros_hbm, out_hbm,
              acc, idx_v, tidx_v, ones_v, semi0, semi1, sems):
    c = lax.axis_index("c")
    s = lax.axis_index("s")
    pltpu.sync_copy(zeros_hbm.at[pl.ds(s * RPT, RPT)],
                    acc.at[pl.ds(s * RPT, RPT)])
    pltpu.sync_copy(ones_hbm, ones_v)
    plsc.subcore_barrier()

    base = c * E + s * EPT
    semi = (semi0, semi1)

    def fire_idx(g, p):
        for b in range(NBUF):
            raw = base + (g * NBUF + b) * CHUNKD
            st = pl.multiple_of(jnp.minimum(raw, 2 * E - CHUNKD), 8)
            pltpu.async_copy(edges_hbm.at[pl.ds(st, CHUNKD)],
                             idx_v.at[p, b], semi[p])

    def drain_scatters(p):
        for b in range(NBUF):
            pltpu.make_async_copy(ones_v, acc.at[idx_v.at[p, b]],
                                  sems).wait()

    def work(g, p, next_g, drain_prev):
        for b in range(NBUF):
            pltpu.make_async_copy(edges_hbm.at[pl.ds(0, CHUNKD)],
                                  idx_v.at[p, b], semi[p]).wait()
        if drain_prev:
            drain_scatters(p)
        if next_g is not None:
            fire_idx(next_g, 1 - p)
        for b in range(NBUF):
            pltpu.async_copy(ones_v, acc.at[idx_v.at[p, b]], sems,
                             add=True)

    fire_idx(0, 0)
    work(0, 0, 1, False)

    def pair(m, carry):
        work(2 * m + 1, 1, 2 * m + 2, True)
        work(2 * m + 2, 0, 2 * m + 3, True)
        return carry

    lax.fori_loop(0, (GRPD2 - 1) // 2 - 1, pair, 0)
    work(GRPD2 - 2, 1, GRPD2 - 1, True)
    work(GRPD2 - 1, 0, None, True)
    drain_scatters(0)
    st = pl.multiple_of(base + CPTD * CHUNKD, 8)
    pltpu.sync_copy(edges_hbm.at[pl.ds(st, TAILD2)], tidx_v)
    pltpu.sync_copy(ones_v.at[pl.ds(0, TAILD2)], acc.at[tidx_v], add=True)

    plsc.subcore_barrier()
    pltpu.sync_copy(acc.at[pl.ds(s * RPT, RPT)],
                    out_hbm.at[c, pl.ds(s * RPT, RPT)])


DEGW = 16

_deg_call = pl.kernel(
    _deg_body,
    out_type=jax.ShapeDtypeStruct((NC, NP, DEGW), jnp.float32),
    mesh=_mesh,
    compiler_params=pltpu.CompilerParams(use_tc_tiling_on_sc=False),
    scratch_types=[
        pltpu.VMEM_SHARED((NP, DEGW), jnp.float32),
        pltpu.VMEM((2, NBUF, CHUNKD), jnp.int32),
        pltpu.VMEM((TAILD2,), jnp.int32),
        pltpu.VMEM((CHUNKD, DEGW), jnp.float32),
        pltpu.SemaphoreType.DMA,
        pltpu.SemaphoreType.DMA,
        pltpu.SemaphoreType.DMA,
    ],
)


def _scatter_body(pre_hbm, src_hbm, dst_hbm, zeros_hbm, out_hbm,
                  acc, idx_s, idx_d, rows,
                  semi0, semi1, semg0, semg1, semg2, semg3, sems):
    c = lax.axis_index("c")
    s = lax.axis_index("s")
    w = s * NC + c
    pltpu.sync_copy(zeros_hbm.at[pl.ds(s * RPT, RPT)],
                    acc.at[pl.ds(s * RPT, RPT)])
    plsc.subcore_barrier()

    base = w * EPW
    semi = (semi0, semi1)
    semg = (semg0, semg1, semg2, semg3)

    def fire_idx(g, p):
        for b in range(NBUF):
            raw = base + (g * NBUF + b) * CHUNK
            st = pl.multiple_of(jnp.minimum(raw, E - CHUNK), 8)
            pltpu.async_copy(src_hbm.at[pl.ds(st, CHUNK)],
                             idx_s.at[p, b], semi[p])
            pltpu.async_copy(dst_hbm.at[pl.ds(st, CHUNK)],
                             idx_d.at[p, b], semi[p])

    def drain_scatters(p):
        for b in range(NBUF):
            pltpu.make_async_copy(rows.at[b], acc.at[idx_d.at[p, b]],
                                  sems).wait()

    def work(g, p, next_g, drain_prev):
        for b in range(NBUF):
            pltpu.make_async_copy(src_hbm.at[pl.ds(0, CHUNK)],
                                  idx_s.at[p, b], semi[p]).wait()
            pltpu.make_async_copy(dst_hbm.at[pl.ds(0, CHUNK)],
                                  idx_d.at[p, b], semi[p]).wait()
        if drain_prev:
            drain_scatters(p)
        if next_g is not None:
            fire_idx(next_g, 1 - p)
        dg = [pltpu.async_copy(pre_hbm.at[idx_s.at[p, b]], rows.at[b],
                               semg[b])
              for b in range(NBUF)]
        for b in range(NBUF):
            dg[b].wait()
            pltpu.async_copy(rows.at[b], acc.at[idx_d.at[p, b]],
                             sems, add=True)

    fire_idx(0, 0)
    work(0, 0, 1, False)

    def pair(m, carry):
        work(2 * m + 1, 1, 2 * m + 2, True)
        work(2 * m + 2, 0, 2 * m + 3, True)
        return carry

    lax.fori_loop(0, GRP // 2 - 1, pair, 0)
    work(GRP - 2, 1, GRP - 1, True)
    work(GRP - 1, 0, None, True)
    drain_scatters(0)
    for t in range(TAIL):
        st = pl.multiple_of(base + (GRP * NBUF + t) * CHUNK, 8)
        pltpu.sync_copy(src_hbm.at[pl.ds(st, CHUNK)], idx_s.at[0, 0])
        pltpu.sync_copy(dst_hbm.at[pl.ds(st, CHUNK)], idx_d.at[0, 0])
        pltpu.async_copy(pre_hbm.at[idx_s.at[0, 0]], rows.at[0],
                         semg[0]).wait()
        pltpu.sync_copy(rows.at[0], acc.at[idx_d.at[0, 0]], add=True)

    plsc.subcore_barrier()
    pltpu.sync_copy(acc.at[pl.ds(s * RPT, RPT)],
                    out_hbm.at[c, pl.ds(s * RPT, RPT)])


_scatter128 = pl.kernel(
    _scatter_body,
    out_type=jax.ShapeDtypeStruct((NC, NP, 128), jnp.float32),
    mesh=_mesh,
    scratch_types=[
        pltpu.VMEM_SHARED((NP, 128), jnp.float32),
        pltpu.VMEM((2, NBUF, CHUNK), jnp.int32),
        pltpu.VMEM((2, NBUF, CHUNK), jnp.int32),
        pltpu.VMEM((NBUF, CHUNK, 128), jnp.float32),
        pltpu.SemaphoreType.DMA,
        pltpu.SemaphoreType.DMA,
        pltpu.SemaphoreType.DMA,
        pltpu.SemaphoreType.DMA,
        pltpu.SemaphoreType.DMA,
        pltpu.SemaphoreType.DMA,
        pltpu.SemaphoreType.DMA,
    ],
)


_scatter64 = pl.kernel(
    _scatter_body,
    out_type=jax.ShapeDtypeStruct((NC, NP, D_OUT), jnp.float32),
    mesh=_mesh,
    compiler_params=pltpu.CompilerParams(use_tc_tiling_on_sc=False),
    scratch_types=[
        pltpu.VMEM_SHARED((NP, D_OUT), jnp.float32),
        pltpu.VMEM((2, NBUF, CHUNK), jnp.int32),
        pltpu.VMEM((2, NBUF, CHUNK), jnp.int32),
        pltpu.VMEM((NBUF, CHUNK, D_OUT), jnp.float32),
        pltpu.SemaphoreType.DMA,
        pltpu.SemaphoreType.DMA,
        pltpu.SemaphoreType.DMA,
        pltpu.SemaphoreType.DMA,
        pltpu.SemaphoreType.DMA,
        pltpu.SemaphoreType.DMA,
        pltpu.SemaphoreType.DMA,
    ],
)


_BM = 1000


def _rs(deg_ref):
    return lax.rsqrt(jnp.maximum(deg_ref[0, :, 0:1], 1.0))


def _deg_spec(k):
    return pl.BlockSpec((1, _BM, DEGW), lambda i: (k, i, 0))


def _mm_scale_body(x_ref, w_ref, dego_ref, o_ref):
    o_ref[...] = jnp.dot(x_ref[...], w_ref[...],
                         preferred_element_type=jnp.float32) * _rs(dego_ref)


def _mm_scale(x, w, degs):
    din, dout = w.shape
    return pl.pallas_call(
        _mm_scale_body,
        grid=(N // _BM,),
        in_specs=[
            pl.BlockSpec((_BM, din), lambda i: (i, 0)),
            pl.BlockSpec((din, dout), lambda i: (0, 0)),
            _deg_spec(0),
        ],
        out_specs=pl.BlockSpec((_BM, dout), lambda i: (i, 0)),
        out_shape=jax.ShapeDtypeStruct((N, dout), jnp.float32),
    )(x, w, degs)


def _combine_mm_body(p_ref, degi_ref, b_ref, w_ref, dego_ref, o_ref):
    h = (p_ref[0] + p_ref[1]) * _rs(degi_ref) + b_ref[...]
    h = jnp.maximum(h, 0.0)
    o_ref[...] = jnp.dot(h, w_ref[...],
                         preferred_element_type=jnp.float32) * _rs(dego_ref)


def _combine_mm(parts, degs, b, w):
    din, dout = w.shape
    return pl.pallas_call(
        _combine_mm_body,
        grid=(N // _BM,),
        in_specs=[
            pl.BlockSpec((NC, _BM, din), lambda i: (0, i, 0)),
            _deg_spec(1),
            pl.BlockSpec((1, din), lambda i: (0, 0)),
            pl.BlockSpec((din, dout), lambda i: (0, 0)),
            _deg_spec(0),
        ],
        out_specs=pl.BlockSpec((_BM, dout), lambda i: (i, 0)),
        out_shape=jax.ShapeDtypeStruct((NP, dout), jnp.float32),
    )(parts, degs, b, w, degs)


def _final_body(p_ref, degi_ref, b_ref, o_ref):
    v = p_ref[0] + p_ref[1]
    o_ref[...] = v * _rs(degi_ref) + b_ref[...]


def _final(parts, degs, b):
    return pl.pallas_call(
        _final_body,
        grid=(N // _BM,),
        in_specs=[
            pl.BlockSpec((NC, _BM, D_OUT), lambda i: (0, i, 0)),
            _deg_spec(1),
            pl.BlockSpec((1, D_OUT), lambda i: (0, 0)),
        ],
        out_specs=pl.BlockSpec((_BM, D_OUT), lambda i: (i, 0)),
        out_shape=jax.ShapeDtypeStruct((N, D_OUT), jnp.float32),
    )(parts, degs, b)


@jax.jit
def kernel(x, edge_index, W1, b1, W2, b2):
    src = edge_index[0]
    dst = edge_index[1]
    edges_flat = edge_index.reshape(-1)

    ones16 = jnp.ones((CHUNKD, DEGW), jnp.float32)
    zeros16 = jnp.zeros((NP, DEGW), jnp.float32)
    zeros128 = jnp.zeros((NP, 128), jnp.float32)
    degs = _deg_call(edges_flat, ones16, zeros16)

    pre1 = _mm_scale(x, W1, degs)
    part1 = _scatter128(pre1, src, dst, zeros128)
    pre2 = _combine_mm(part1, degs, b1.reshape(1, D_HID), W2)
    zeros64 = jnp.zeros((NP, D_OUT), jnp.float32)
    part2 = _scatter64(pre2, src, dst, zeros64)
    return _final(part2, degs, b2.reshape(1, D_OUT))
